# baseline (speedup 1.0000x reference)
import functools

import jax
import jax.numpy as jnp
from jax.experimental import pallas as pl
from jax.experimental.pallas import tpu as pltpu

_THRESHOLD = 0.6
_TEMPERATURE = 0.04
_EPS = 1e-06
_TWELVE_BITS = 0x41400000
_BISECT_ITERS = 24


def _softplus(x):
    return jnp.maximum(x, 0.0) + jnp.log1p(jnp.exp(-jnp.abs(x)))


def _body(att_ref, ir_ref, vis_ref, out_ref, acc_scr, lg_scr, *, H):
    b = pl.program_id(0)
    hb = pl.program_id(1)

    @pl.when((b == 0) & (hb == 0))
    def _():
        out_ref[...] = jnp.zeros_like(out_ref)

    hsum = jnp.sum(att_ref[0], axis=0)

    @pl.when(hb == 0)
    def _():
        acc_scr[...] = hsum
        v = vis_ref[0]
        vn = v / (jnp.sqrt(jnp.sum(v * v, axis=-1, keepdims=True)) + _EPS)
        irb = ir_ref[0]
        irn = irb / ((jnp.sqrt(jnp.sum(irb * irb, axis=-1, keepdims=True))
                      + _EPS) * _TEMPERATURE)
        lg = jax.lax.dot_general(
            vn, irn, (((1,), (1,)), ((), ())),
            preferred_element_type=jnp.float32,
        )
        lg_scr[...] = lg
        out_ref[...] += jnp.reshape(jnp.sum(_softplus(lg)), (1, 1))

    @pl.when(hb == 1)
    def _():
        am12_t = jnp.transpose(acc_scr[...] + hsum)

        thr = _THRESHOLD * H
        N = am12_t.shape[1]
        lo = jnp.zeros((1, N), jnp.int32)
        hi = jnp.full((1, N), _TWELVE_BITS, jnp.int32)
        for _ in range(_BISECT_ITERS):
            mid = (lo + hi) >> 1
            midf = jax.lax.bitcast_convert_type(mid, jnp.float32)
            s = jnp.sum(jnp.where(am12_t >= midf, am12_t, 0.0), axis=0,
                        keepdims=True)
            take = s <= thr
            lo = jnp.where(take, lo, mid)
            hi = jnp.where(take, mid, hi)
        hif = jax.lax.bitcast_convert_type(hi, jnp.float32)

        masked = jnp.where(am12_t >= hif, lg_scr[...], 0.0)
        out_ref[...] += jnp.reshape(-jnp.sum(masked), (1, 1))


def kernel(vis_embeds, ir_embeds, attention_map):
    B, H, N, M = attention_map.shape
    D = vis_embeds.shape[-1]
    grid = (B, 2)

    total = pl.pallas_call(
        functools.partial(_body, H=H),
        grid=grid,
        in_specs=[
            pl.BlockSpec((1, H // 2, N, M), lambda b, hb: (b, hb, 0, 0)),
            pl.BlockSpec((1, N, D), lambda b, hb: (b, 0, 0)),
            pl.BlockSpec((1, M, D), lambda b, hb: (b, 0, 0)),
        ],
        out_specs=pl.BlockSpec((1, 1), lambda b, hb: (0, 0)),
        out_shape=jax.ShapeDtypeStruct((1, 1), jnp.float32),
        scratch_shapes=[
            pltpu.VMEM((N, M), jnp.float32),
            pltpu.VMEM((M, N), jnp.float32),
        ],
    )(attention_map, ir_embeds, vis_embeds)
    return (total[0, 0] / (B * N * M)).astype(jnp.float32)


# speedup vs baseline: 1.3453x; 1.3453x over previous
import functools

import jax
import jax.numpy as jnp
from jax.experimental import pallas as pl
from jax.experimental.pallas import tpu as pltpu

_THRESHOLD = 0.6
_TEMPERATURE = 0.04
_EPS = 1e-06
_TWELVE_BITS = 0x41400000
_BISECT_ITERS = 20


def _softplus(x):
    return jnp.maximum(x, 0.0) + jnp.log1p(jnp.exp(-jnp.abs(x)))


def _body(att_ref, ir_ref, vis_ref, out_ref, *, H):
    b = pl.program_id(0)

    @pl.when(b == 0)
    def _():
        out_ref[...] = jnp.zeros_like(out_ref)

    am12 = jnp.sum(att_ref[0], axis=0)
    am12_t = jnp.transpose(am12)

    v = vis_ref[0]
    vn = v / (jnp.sqrt(jnp.sum(v * v, axis=-1, keepdims=True)) + _EPS)
    irb = ir_ref[0]
    irn = irb / ((jnp.sqrt(jnp.sum(irb * irb, axis=-1, keepdims=True)) + _EPS)
                 * _TEMPERATURE)
    lg = jax.lax.dot_general(
        vn, irn, (((1,), (1,)), ((), ())),
        preferred_element_type=jnp.float32,
    )

    thr = _THRESHOLD * H
    N = am12_t.shape[1]
    lo = jnp.zeros((1, N), jnp.int32)
    hi = jnp.full((1, N), _TWELVE_BITS, jnp.int32)
    for _ in range(_BISECT_ITERS):
        mid = (lo + hi) >> 1
        midf = jax.lax.bitcast_convert_type(mid, jnp.float32)
        s = jnp.sum(jnp.where(am12_t >= midf, am12_t, 0.0), axis=0,
                    keepdims=True)
        take = s <= thr
        lo = jnp.where(take, lo, mid)
        hi = jnp.where(take, mid, hi)
    hif = jax.lax.bitcast_convert_type(hi, jnp.float32)

    masked = jnp.where(am12_t >= hif, lg, 0.0)
    bsum = jnp.sum(_softplus(lg)) - jnp.sum(masked)
    out_ref[...] += jnp.reshape(bsum, (1, 1))


def kernel(vis_embeds, ir_embeds, attention_map):
    B, H, N, M = attention_map.shape
    D = vis_embeds.shape[-1]
    grid = (B,)

    total = pl.pallas_call(
        functools.partial(_body, H=H),
        grid=grid,
        in_specs=[
            pl.BlockSpec((1, H, N, M), lambda b: (b, 0, 0, 0)),
            pl.BlockSpec((1, N, D), lambda b: (b, 0, 0)),
            pl.BlockSpec((1, M, D), lambda b: (b, 0, 0)),
        ],
        out_specs=pl.BlockSpec((1, 1), lambda b: (0, 0)),
        out_shape=jax.ShapeDtypeStruct((1, 1), jnp.float32),
    )(attention_map, ir_embeds, vis_embeds)
    return (total[0, 0] / (B * N * M)).astype(jnp.float32)


# speedup vs baseline: 1.3465x; 1.0009x over previous
import functools

import jax
import jax.numpy as jnp
from jax.experimental import pallas as pl

_THRESHOLD = 0.6
_TEMPERATURE = 0.04
_EPS = 1e-06
_TWELVE_BITS = 0x41400000
_BISECT_ITERS = 20


def _softplus(x):
    return jnp.maximum(x, 0.0) + jnp.log1p(jnp.exp(-jnp.abs(x)))


def _body(att_ref, ir_ref, vis_ref, out_ref, *, H):
    b = pl.program_id(0)

    @pl.when(b == 0)
    def _():
        out_ref[...] = jnp.zeros_like(out_ref)

    am12 = jnp.sum(att_ref[0], axis=0)
    am12_t = jnp.transpose(am12)

    v = vis_ref[0]
    vn = v / (jnp.sqrt(jnp.sum(v * v, axis=-1, keepdims=True)) + _EPS)
    irb = ir_ref[0]
    irn = irb / ((jnp.sqrt(jnp.sum(irb * irb, axis=-1, keepdims=True)) + _EPS)
                 * _TEMPERATURE)
    lg = jax.lax.dot_general(
        vn, irn, (((1,), (1,)), ((), ())),
        preferred_element_type=jnp.float32,
    )

    thr = _THRESHOLD * H
    N = am12_t.shape[1]
    lo = jnp.zeros((1, N), jnp.int32)
    hi = jnp.full((1, N), _TWELVE_BITS, jnp.int32)
    for _ in range(_BISECT_ITERS):
        mid = (lo + hi) >> 1
        midf = jax.lax.bitcast_convert_type(mid, jnp.float32)
        s = jnp.sum(jnp.where(am12_t >= midf, am12_t, 0.0), axis=0,
                    keepdims=True)
        take = s <= thr
        lo = jnp.where(take, lo, mid)
        hi = jnp.where(take, mid, hi)
    hif = jax.lax.bitcast_convert_type(hi, jnp.float32)

    masked = jnp.where(am12_t >= hif, lg, 0.0)
    bsum = jnp.sum(_softplus(lg)) - jnp.sum(masked)
    out_ref[...] += jnp.reshape(bsum, (1, 1))


def kernel(vis_embeds, ir_embeds, attention_map):
    B, H, N, M = attention_map.shape
    D = vis_embeds.shape[-1]
    grid = (B,)

    total = pl.pallas_call(
        functools.partial(_body, H=H),
        grid=grid,
        in_specs=[
            pl.BlockSpec((1, H, N, M), lambda b: (b, 0, 0, 0)),
            pl.BlockSpec((1, N, D), lambda b: (b, 0, 0)),
            pl.BlockSpec((1, M, D), lambda b: (b, 0, 0)),
        ],
        out_specs=pl.BlockSpec((1, 1), lambda b: (0, 0)),
        out_shape=jax.ShapeDtypeStruct((1, 1), jnp.float32),
    )(attention_map, ir_embeds, vis_embeds)
    return (total[0, 0] / (B * N * M)).astype(jnp.float32)
